# baseline (speedup 1.0000x reference)
import functools

import jax
import jax.numpy as jnp
from jax import lax
from jax.experimental import pallas as pl
from jax.experimental.pallas import tpu as pltpu


def _bn_onepass_kernel(x_ref, gamma_ref, beta_ref, o_ref, *, inv_count, eps):
    xf = x_ref[...]
    s_bt = jnp.sum(xf, axis=0)
    sq_bt = jnp.sum(xf * xf, axis=0)
    s = jnp.sum(s_bt, axis=1, keepdims=True)
    sq = jnp.sum(sq_bt, axis=1, keepdims=True)
    mean = s * inv_count
    var = jnp.maximum(sq * inv_count - mean * mean, 0.0)
    inv_std = lax.rsqrt(var + eps)
    scale = gamma_ref[...] * inv_std
    shift = beta_ref[...] - mean * scale
    o_ref[...] = xf * scale[None] + shift[None]


def kernel(x, gamma, beta, *, eps=1e-5):
    shape = x.shape
    x3 = x.reshape(shape[0], -1, shape[-1])
    Bn, N, T = x3.shape

    gamma2 = gamma.reshape(N, 1).astype(jnp.float32)
    beta2 = beta.reshape(N, 1).astype(jnp.float32)
    inv_count = 1.0 / float(Bn * T)

    itemsize = jnp.dtype(x3.dtype).itemsize
    n_blk = N
    while n_blk > 8 and Bn * n_blk * T * itemsize > (2 << 20):
        n_blk //= 2
    n_blk = max(n_blk, 8)
    Np = pl.cdiv(N, n_blk) * n_blk
    if Np != N:
        x3 = jnp.pad(x3, ((0, 0), (0, Np - N), (0, 0)))
        gamma2 = jnp.pad(gamma2, ((0, Np - N), (0, 0)))
        beta2 = jnp.pad(beta2, ((0, Np - N), (0, 0)))

    grid = (Np // n_blk,)
    x_spec = pl.BlockSpec((Bn, n_blk, T), lambda i: (0, i, 0))
    f_spec = pl.BlockSpec((n_blk, 1), lambda i: (i, 0))

    out3 = pl.pallas_call(
        functools.partial(_bn_onepass_kernel, inv_count=inv_count,
                          eps=float(eps)),
        out_shape=jax.ShapeDtypeStruct((Bn, Np, T), x3.dtype),
        grid=grid,
        in_specs=[x_spec, f_spec, f_spec],
        out_specs=x_spec,
        compiler_params=pltpu.CompilerParams(
            dimension_semantics=("parallel",),
            vmem_limit_bytes=64 << 20),
    )(x3, gamma2, beta2)

    if Np != N:
        out3 = out3[:, :N, :]
    return out3.reshape(shape)


# speedup vs baseline: 1.1708x; 1.1708x over previous
import functools

import jax
import jax.numpy as jnp
from jax import lax
from jax.experimental import pallas as pl
from jax.experimental.pallas import tpu as pltpu


def _bn_onepass_kernel(x_ref, gamma_ref, beta_ref, o_ref, *, inv_count, eps):
    xf = x_ref[...]
    s_bt = jnp.sum(xf, axis=0)
    sq_bt = jnp.sum(xf * xf, axis=0)
    s = jnp.sum(s_bt, axis=1, keepdims=True)
    sq = jnp.sum(sq_bt, axis=1, keepdims=True)
    mean = s * inv_count
    var = jnp.maximum(sq * inv_count - mean * mean, 0.0)
    inv_std = lax.rsqrt(var + eps)
    scale = gamma_ref[...] * inv_std
    shift = beta_ref[...] - mean * scale
    o_ref[...] = xf * scale[None] + shift[None]


def kernel(x, gamma, beta, *, eps=1e-5):
    shape = x.shape
    x3 = x.reshape(shape[0], -1, shape[-1])
    Bn, N, T = x3.shape

    gamma2 = gamma.reshape(N, 1).astype(jnp.float32)
    beta2 = beta.reshape(N, 1).astype(jnp.float32)
    inv_count = 1.0 / float(Bn * T)

    itemsize = jnp.dtype(x3.dtype).itemsize
    n_blk = N
    while n_blk > 8 and Bn * n_blk * T * itemsize > (8 << 20):
        n_blk //= 2
    n_blk = max(n_blk, 8)
    Np = pl.cdiv(N, n_blk) * n_blk
    if Np != N:
        x3 = jnp.pad(x3, ((0, 0), (0, Np - N), (0, 0)))
        gamma2 = jnp.pad(gamma2, ((0, Np - N), (0, 0)))
        beta2 = jnp.pad(beta2, ((0, Np - N), (0, 0)))

    grid = (Np // n_blk,)
    x_spec = pl.BlockSpec((Bn, n_blk, T), lambda i: (0, i, 0))
    f_spec = pl.BlockSpec((n_blk, 1), lambda i: (i, 0))

    out3 = pl.pallas_call(
        functools.partial(_bn_onepass_kernel, inv_count=inv_count,
                          eps=float(eps)),
        out_shape=jax.ShapeDtypeStruct((Bn, Np, T), x3.dtype),
        grid=grid,
        in_specs=[x_spec, f_spec, f_spec],
        out_specs=x_spec,
        compiler_params=pltpu.CompilerParams(
            dimension_semantics=("parallel",),
            vmem_limit_bytes=64 << 20),
    )(x3, gamma2, beta2)

    if Np != N:
        out3 = out3[:, :N, :]
    return out3.reshape(shape)
